# Initial kernel scaffold; baseline (speedup 1.0000x reference)
import jax
import jax.numpy as jnp
from jax.experimental import pallas as pl


def kernel(x):
    raise NotImplementedError("write your pallas kernel here")


# speedup vs baseline: 81.2381x; 81.2381x over previous
import functools

import numpy as np
import jax
import jax.numpy as jnp
from jax import lax
from jax.experimental import pallas as pl

_HEIGHT = 128
_MAX_SCALE = 3.5
_KS = 31
_PAD = _KS // 2
_EPS = 1e-8
_B, _S = 64, 4096


def _build_table() -> np.ndarray:
    xs = np.arange(_KS, dtype=np.float32) - _KS // 2
    g = np.exp(-(xs ** 2) / np.float32(2.0)).astype(np.float32)
    g = (g / g.sum()).astype(np.float32)

    def refl(p):
        if p < 0:
            return -p
        if p > _HEIGHT - 1:
            return 2 * (_HEIGHT - 1) - p
        return p

    pidx = [refl(p) for p in range(-_PAD, _HEIGHT + _PAD)]
    tt = np.zeros((_HEIGHT, _HEIGHT), np.float32)
    for h in range(_HEIGHT):
        for k in range(_KS):
            tt[h, pidx[h + k]] += g[k]
    return tt


_TT = _build_table()


def _tc_body(x_ref, tt_ref, o_ref):
    row = x_ref[0, 0, :]
    n = row.shape[0]
    mean = jnp.sum(row) / n
    xc = row - mean
    var = jnp.sum(xc * xc) / (n - 1)
    std = jnp.sqrt(var) + _EPS
    xn = jnp.clip(xc / std, -_MAX_SCALE, _MAX_SCALE)
    binf = (xn + _MAX_SCALE) / (2.0 * _MAX_SCALE) * _HEIGHT
    bins = jnp.clip(binf.astype(jnp.int32), 0, _HEIGHT - 1)
    hh = lax.broadcasted_iota(jnp.int32, (_HEIGHT, n), 0)
    onehot = (hh == bins[None, :]).astype(jnp.float32)
    o_ref[0] = jnp.dot(tt_ref[...], onehot, preferred_element_type=jnp.float32)


@jax.jit
def kernel(x):
    out = pl.pallas_call(
        _tc_body,
        grid=(_B,),
        in_specs=[
            pl.BlockSpec((1, 1, _S), lambda b: (b, 0, 0)),
            pl.BlockSpec((_HEIGHT, _HEIGHT), lambda b: (0, 0)),
        ],
        out_specs=pl.BlockSpec((1, _HEIGHT, _S), lambda b: (b, 0, 0)),
        out_shape=jax.ShapeDtypeStruct((_B, _HEIGHT, _S), jnp.float32),
    )(x.reshape(_B, 1, _S), jnp.asarray(_TT))
    return out[:, None]
